# Initial kernel scaffold; baseline (speedup 1.0000x reference)
import jax
import jax.numpy as jnp
from jax.experimental import pallas as pl


def kernel(x, edge_index, batch, W1, a_src1, a_dst1, b1, W2, a_src2, a_dst2, b2, Wfc, bfc):
    raise NotImplementedError("write your pallas kernel here")


# speedup vs baseline: 3.6741x; 3.6741x over previous
import dataclasses
import functools

import jax
import jax.numpy as jnp
from jax import lax
from jax.experimental import pallas as pl
from jax.experimental.pallas import tpu as pltpu
from jax.experimental.pallas import tpu_sc as plsc

N = 10000
E = 160000
EN = E + N
F_IN = 256
H1 = 10
C1 = 256
OUT = 128
B = 512

NPAD = 10240
NT = 1024
NW = 32
R = 320
L = 16

CH0 = 34000
FLUSH = 1024
BE = 2016
G = 48
BINCAP = 173056

_MESH = dict(core_axis_name="c", subcore_axis_name="s")

_SC_PARAMS = pltpu.CompilerParams()
if "needs_layout_passes" in pltpu.CompilerParams.__dataclass_fields__:
  _SC_PARAMS = dataclasses.replace(_SC_PARAMS, needs_layout_passes=False)


def _wid():
  return lax.axis_index("c") * 16 + lax.axis_index("s")


def _al(v):
  return pl.multiple_of(v, L)


def _tc1(x_pad, w1r, as1, ad1):
  def body(x_ref, w_ref, as_ref, ad_ref, h_ref, als_ref, ald_ref):
    hh = jnp.dot(x_ref[...], w_ref[0], preferred_element_type=jnp.float32)
    h_ref[0] = hh
    als_ref[0, 0] = jnp.sum(hh * as_ref[0], axis=1)
    ald_ref[0, 0] = jnp.sum(hh * ad_ref[0], axis=1)

  return pl.pallas_call(
      body,
      grid=(H1, NPAD // NT),
      in_specs=[
          pl.BlockSpec((NT, F_IN), lambda h, n: (n, 0)),
          pl.BlockSpec((1, F_IN, C1), lambda h, n: (h, 0, 0)),
          pl.BlockSpec((1, 1, C1), lambda h, n: (h, 0, 0)),
          pl.BlockSpec((1, 1, C1), lambda h, n: (h, 0, 0)),
      ],
      out_specs=[
          pl.BlockSpec((1, NT, C1), lambda h, n: (h, n, 0)),
          pl.BlockSpec((1, 1, NT), lambda h, n: (h, 0, n)),
          pl.BlockSpec((1, 1, NT), lambda h, n: (h, 0, n)),
      ],
      out_shape=[
          jax.ShapeDtypeStruct((H1, NPAD, C1), jnp.float32),
          jax.ShapeDtypeStruct((H1, 1, NPAD), jnp.float32),
          jax.ShapeDtypeStruct((H1, 1, NPAD), jnp.float32),
      ],
  )(x_pad, w1r, as1, ad1)


def _tc2(msg1, den1, b1r, w2r, as2, ad2):
  def body(m_ref, d_ref, b_ref, w_ref, s_ref, t_ref, h2_ref, als_ref, ald_ref):
    acc = jnp.zeros((NT, OUT), jnp.float32)
    for h in range(H1):
      dh = d_ref[h, 0, :][:, None] + 1e-16
      hv = m_ref[h] / dh + b_ref[h]
      hp = jnp.where(hv > 0, hv, jnp.exp(jnp.minimum(hv, 0.0)) - 1.0)
      acc = acc + jnp.dot(hp, w_ref[h], preferred_element_type=jnp.float32)
    h2_ref[...] = acc
    als_ref[0, 0] = jnp.sum(acc * s_ref[0], axis=1)
    ald_ref[0, 0] = jnp.sum(acc * t_ref[0], axis=1)

  return pl.pallas_call(
      body,
      grid=(NPAD // NT,),
      in_specs=[
          pl.BlockSpec((H1, NT, C1), lambda n: (0, n, 0)),
          pl.BlockSpec((H1, 1, NT), lambda n: (0, 0, n)),
          pl.BlockSpec((H1, 1, C1), lambda n: (0, 0, 0)),
          pl.BlockSpec((H1, C1, OUT), lambda n: (0, 0, 0)),
          pl.BlockSpec((1, 1, OUT), lambda n: (0, 0, 0)),
          pl.BlockSpec((1, 1, OUT), lambda n: (0, 0, 0)),
      ],
      out_specs=[
          pl.BlockSpec((NT, OUT), lambda n: (n, 0)),
          pl.BlockSpec((1, 1, NT), lambda n: (0, 0, n)),
          pl.BlockSpec((1, 1, NT), lambda n: (0, 0, n)),
      ],
      out_shape=[
          jax.ShapeDtypeStruct((NPAD, OUT), jnp.float32),
          jax.ShapeDtypeStruct((1, 1, NPAD), jnp.float32),
          jax.ShapeDtypeStruct((1, 1, NPAD), jnp.float32),
      ],
  )(msg1, den1, b1r, w2r, as2, ad2)


def _tc3(msg2, den2, b2r):
  def body(m_ref, d_ref, b_ref, o_ref):
    o_ref[...] = jnp.maximum(
        m_ref[...] / (d_ref[0, 0, :][:, None] + 1e-16) + b_ref[...], 0.0)

  return pl.pallas_call(
      body,
      grid=(NPAD // NT,),
      in_specs=[
          pl.BlockSpec((NT, OUT), lambda n: (n, 0)),
          pl.BlockSpec((1, 1, NT), lambda n: (0, 0, n)),
          pl.BlockSpec((1, OUT), lambda n: (0, 0)),
      ],
      out_specs=pl.BlockSpec((NT, OUT), lambda n: (n, 0)),
      out_shape=jax.ShapeDtypeStruct((NPAD, OUT), jnp.float32),
  )(msg2, den2, b2r)


def _tc4(pooled, wfc, bfcr):
  def body(p_ref, w_ref, b_ref, o_ref):
    o_ref[...] = jnp.maximum(
        jnp.dot(p_ref[...], w_ref[...], preferred_element_type=jnp.float32)
        + b_ref[...], 0.0)

  return pl.pallas_call(
      body,
      out_shape=jax.ShapeDtypeStruct((B, OUT), jnp.float32),
  )(pooled, wfc, bfcr)


def _sc_bin_count(d_full):
  @functools.partial(
      pl.kernel,
      out_type=jax.ShapeDtypeStruct((NW, L), jnp.int32),
      mesh=plsc.VectorSubcoreMesh(**_MESH),
      compiler_params=_SC_PARAMS,
      scratch_types=[
          pltpu.VMEM((CH0,), jnp.int32),
          pltpu.VMEM((L,), jnp.int32),
          pltpu.SemaphoreType.DMA,
      ],
  )
  def k(d_hbm, counts_hbm, dbuf, cbuf, sem):
    wid = _wid()
    lo = wid * R
    hi = lo + R

    def chunk(ci, cnt):
      pltpu.sync_copy(d_hbm.at[pl.ds(ci * CH0, CH0)], dbuf)

      def vec(vi, cnt):
        d16 = dbuf[pl.ds(vi * L, L)]
        m = (d16 >= lo) & (d16 < hi)
        return cnt + jnp.where(m, 1, 0).astype(jnp.int32)

      return lax.fori_loop(0, CH0 // L, vec, cnt)

    cnt = lax.fori_loop(0, EN // CH0, chunk, jnp.zeros((L,), jnp.int32))
    total = jnp.sum(cnt)
    cbuf[...] = jnp.full((L,), total, jnp.int32)
    pltpu.sync_copy(cbuf, counts_hbm.at[wid])

  return k(d_full)


def _prefix_from_counts(cbuf, wid):
  my_start = jnp.int32(0)
  total = jnp.int32(0)
  my_count = jnp.int32(0)
  for rr in range(NW):
    c = jnp.max(cbuf[rr])
    cpad = ((c + L - 1) // L) * L
    my_start = my_start + jnp.where(rr < wid, cpad, 0)
    my_count = my_count + jnp.where(rr == wid, c, 0)
    total = total + cpad
  return my_start, my_count, total


def _sc_bin_fill(d_full, s_full, counts):
  @functools.partial(
      pl.kernel,
      out_type=[
          jax.ShapeDtypeStruct((BINCAP,), jnp.int32),
          jax.ShapeDtypeStruct((BINCAP,), jnp.int32),
      ],
      mesh=plsc.VectorSubcoreMesh(**_MESH),
      compiler_params=_SC_PARAMS,
      scratch_types=[
          pltpu.VMEM((CH0,), jnp.int32),
          pltpu.VMEM((CH0,), jnp.int32),
          pltpu.VMEM((FLUSH + L,), jnp.int32),
          pltpu.VMEM((FLUSH + L,), jnp.int32),
          pltpu.VMEM((NW, L), jnp.int32),
          pltpu.SemaphoreType.DMA,
      ],
  )
  def k(d_hbm, s_hbm, c_hbm, sb_hbm, db_hbm, dbuf, sbuf, sts, std, cbuf, sem):
    wid = _wid()
    lo = wid * R
    hi = lo + R
    pltpu.sync_copy(c_hbm, cbuf)
    my_start, my_count, total = _prefix_from_counts(cbuf, wid)

    zero = jnp.zeros((L,), jnp.int32)
    for z in range((FLUSH + L) // L):
      sts[pl.ds(z * L, L)] = zero
      std[pl.ds(z * L, L)] = zero

    def chunk(ci, carry):
      pltpu.sync_copy(d_hbm.at[pl.ds(ci * CH0, CH0)], dbuf)
      pltpu.sync_copy(s_hbm.at[pl.ds(ci * CH0, CH0)], sbuf)

      def vec(vi, carry):
        staged, cursor = carry
        d16 = dbuf[pl.ds(vi * L, L)]
        s16 = sbuf[pl.ds(vi * L, L)]
        m = (d16 >= lo) & (d16 < hi)
        c = jnp.sum(jnp.where(m, 1, 0).astype(jnp.int32))
        plsc.store_compressed(sts.at[pl.ds(staged, L)], s16, mask=m)
        plsc.store_compressed(std.at[pl.ds(staged, L)], d16 - lo, mask=m)
        staged = staged + c

        def flush(carry):
          staged, cursor = carry
          pltpu.sync_copy(sts.at[pl.ds(0, FLUSH)],
                          sb_hbm.at[pl.ds(_al(cursor), FLUSH)])
          pltpu.sync_copy(std.at[pl.ds(0, FLUSH)],
                          db_hbm.at[pl.ds(_al(cursor), FLUSH)])
          rs = sts[pl.ds(FLUSH, L)]
          rd = std[pl.ds(FLUSH, L)]
          sts[pl.ds(0, L)] = rs
          std[pl.ds(0, L)] = rd
          return staged - FLUSH, cursor + FLUSH

        return lax.cond(staged >= FLUSH, flush, lambda c: c, (staged, cursor))

      return lax.fori_loop(0, CH0 // L, vec, carry)

    staged, cursor = lax.fori_loop(0, EN // CH0, chunk, (jnp.int32(0), my_start))

    ngr = (staged + L - 1) // L

    def drain(gi, _):
      pltpu.sync_copy(sts.at[pl.ds(gi * L, L)],
                      sb_hbm.at[pl.ds(_al(cursor + gi * L), L)])
      pltpu.sync_copy(std.at[pl.ds(gi * L, L)],
                      db_hbm.at[pl.ds(_al(cursor + gi * L), L)])
      return 0

    lax.fori_loop(0, ngr, drain, 0)

    @pl.when(wid == NW - 1)
    def _():
      for z in range(FLUSH // L):
        sts[pl.ds(z * L, L)] = zero
      pltpu.sync_copy(sts.at[pl.ds(0, FLUSH)],
                      sb_hbm.at[pl.ds(_al(total), FLUSH)])
      pltpu.sync_copy(sts.at[pl.ds(0, FLUSH)],
                      sb_hbm.at[pl.ds(_al(total + FLUSH), FLUSH)])
      pltpu.sync_copy(sts.at[pl.ds(0, FLUSH)],
                      db_hbm.at[pl.ds(_al(total), FLUSH)])
      pltpu.sync_copy(sts.at[pl.ds(0, FLUSH)],
                      db_hbm.at[pl.ds(_al(total + FLUSH), FLUSH)])

  return k(d_full, s_full, counts)


def _sc_layer(heads, width, alpha_sT, alpha_dT, rows_tab, counts, sbin, dbin):
  jc = width // L

  @functools.partial(
      pl.kernel,
      out_type=[
          jax.ShapeDtypeStruct((heads, NPAD, width), jnp.float32),
          jax.ShapeDtypeStruct((heads * NPAD,), jnp.float32),
      ],
      mesh=plsc.VectorSubcoreMesh(**_MESH),
      compiler_params=_SC_PARAMS,
      scratch_types=[
          pltpu.VMEM((NPAD,), jnp.float32),
          pltpu.VMEM((R,), jnp.float32),
          pltpu.VMEM((R, width), jnp.float32),
          pltpu.VMEM((G, width), jnp.float32),
          pltpu.VMEM((G, width), jnp.float32),
          pltpu.VMEM((BE,), jnp.int32),
          pltpu.VMEM((BE,), jnp.int32),
          pltpu.VMEM((R,), jnp.float32),
          pltpu.VMEM((NW, L), jnp.int32),
          pltpu.SemaphoreType.DMA,
          pltpu.SemaphoreType.DMA,
          pltpu.SemaphoreType.DMA,
      ],
  )
  def k(as_hbm, ad_hbm, tab_hbm, c_hbm, sb_hbm, db_hbm, msg_hbm, den_hbm,
        table, adr, acc, rowsA, rowsB, sblk, dblk, dn, cbuf, semA, semB, sem):
    wid = _wid()
    lo = wid * R
    pltpu.sync_copy(c_hbm, cbuf)
    my_start, my_count, _ = _prefix_from_counts(cbuf, wid)
    nblk = (my_count + BE - 1) // BE
    iota = lax.iota(jnp.int32, L)

    def gather(h, c, dst, dsem):
      return pltpu.make_async_copy(
          tab_hbm.at[h].at[sblk.at[pl.ds(c * G, G)]], dst, dsem)

    def per_head(h):
      pltpu.sync_copy(as_hbm.at[pl.ds(_al(h * NPAD), NPAD)], table)
      pltpu.sync_copy(ad_hbm.at[pl.ds(_al(h * NPAD + lo), R)], adr)

      def tmax(i, m):
        return jnp.maximum(m, table[pl.ds(i * L, L)])

      gmax = jnp.max(lax.fori_loop(0, N // L, tmax,
                                   jnp.full((L,), -3e38, jnp.float32)))

      zf = jnp.zeros((L,), jnp.float32)

      def zrow(r, _):
        for j in range(jc):
          acc[r, pl.ds(j * L, L)] = zf
        return 0

      lax.fori_loop(0, R, zrow, 0)
      for z in range(R // L):
        dn[pl.ds(z * L, L)] = zf

      def per_block(bi):
        eb0 = bi * BE
        ebn = jnp.minimum(BE, my_count - eb0)
        pltpu.sync_copy(sb_hbm.at[pl.ds(_al(my_start + eb0), BE)], sblk)
        pltpu.sync_copy(db_hbm.at[pl.ds(_al(my_start + eb0), BE)], dblk)
        nch = (ebn + G - 1) // G
        gather(h, 0, rowsA, semA).start()

        def process(c, rows):
          @pl.loop(0, G // L)
          def _(g2):
            base = c * G + g2 * L
            s16 = sblk[pl.ds(base, L)]
            dl16 = dblk[pl.ds(base, L)]
            as16 = plsc.load_gather(table, [s16])
            ad16 = plsc.load_gather(adr, [dl16])
            e16 = as16 + ad16
            e16 = jnp.where(e16 > 0, e16, 0.2 * e16)
            t16 = gmax + ad16
            m16 = jnp.where(t16 > 0, t16, 0.2 * t16)
            w16 = jnp.exp(e16 - m16)
            w16 = jnp.where(base + iota < ebn, w16, 0.0)
            plsc.addupdate_scatter(dn, [dl16], w16)
            for li in range(L):
              w_s = w16[li]
              dl_s = dl16[li]
              for j in range(jc):
                acc[dl_s, pl.ds(j * L, L)] += w_s * rows[g2 * L + li,
                                                         pl.ds(j * L, L)]

        def pair(kk, _):
          c0 = 2 * kk
          c1 = c0 + 1

          @pl.when(c1 < nch)
          def _():
            gather(h, c1, rowsB, semB).start()

          gather(h, c0, rowsA, semA).wait()
          process(c0, rowsA)

          @pl.when(c1 < nch)
          def _():
            @pl.when(c1 + 1 < nch)
            def _():
              gather(h, c1 + 1, rowsA, semA).start()

            gather(h, c1, rowsB, semB).wait()
            process(c1, rowsB)

          return 0

        lax.fori_loop(0, (nch + 1) // 2, pair, 0)

      @pl.loop(0, nblk)
      def _(bi):
        per_block(bi)

      pltpu.sync_copy(acc, msg_hbm.at[h].at[pl.ds(_al(lo), R)])
      pltpu.sync_copy(dn, den_hbm.at[pl.ds(_al(h * NPAD + lo), R)])

    @pl.loop(0, heads)
    def _(h):
      per_head(h)

  return k(alpha_sT, alpha_dT, rows_tab, counts, sbin, dbin)


def _sc_pool(rows2, batch):
  PC = 256
  SEG = B // NW

  @functools.partial(
      pl.kernel,
      out_type=jax.ShapeDtypeStruct((B, OUT), jnp.float32),
      mesh=plsc.VectorSubcoreMesh(**_MESH),
      compiler_params=_SC_PARAMS,
      scratch_types=[
          pltpu.VMEM((N + PC,), jnp.int32),
          pltpu.VMEM((PC * OUT,), jnp.float32),
          pltpu.VMEM((SEG, OUT), jnp.float32),
          pltpu.SemaphoreType.DMA,
      ],
  )
  def k(rows_hbm, b_hbm, out_hbm, bbuf, rbuf, pacc, sem):
    wid = _wid()
    seg0 = wid * SEG
    zi = jnp.zeros((L,), jnp.int32)
    for z in range(PC // L):
      bbuf[pl.ds(N + z * L, L)] = zi
    pltpu.sync_copy(b_hbm, bbuf.at[pl.ds(0, N)])

    def cnt(i, carry):
      cl, chh = carry
      b16 = bbuf[pl.ds(i * L, L)]
      cl = cl + jnp.where(b16 < seg0, 1, 0).astype(jnp.int32)
      chh = chh + jnp.where(b16 < seg0 + SEG, 1, 0).astype(jnp.int32)
      return cl, chh

    cl, chh = lax.fori_loop(0, N // L, cnt, (zi, zi))
    lo = jnp.sum(cl)
    hi = jnp.sum(chh)

    neg = jnp.full((L,), -1.0, jnp.float32)
    for s in range(SEG):
      for j in range(OUT // L):
        pacc[s, pl.ds(j * L, L)] = neg

    nck = (hi - lo + PC - 1) // PC

    @pl.loop(0, nck)
    def _(ck):
      row0 = jnp.minimum(lo + ck * PC, NPAD - PC)
      pltpu.sync_copy(rows_hbm.at[pl.ds(_al(row0 * OUT), PC * OUT)], rbuf)

      @pl.loop(0, PC // L)
      def _(g):
        b16 = jnp.clip(bbuf[pl.ds(row0 + g * L, L)] - seg0, 0, SEG - 1)
        for li in range(L):
          rr = g * L + li
          valid = (row0 + rr >= lo) & (row0 + rr < hi)
          sg = b16[li]
          for j in range(OUT // L):
            v = jnp.where(valid, rbuf[pl.ds(rr * OUT + j * L, L)], -1.0)
            pacc[sg, pl.ds(j * L, L)] = jnp.maximum(pacc[sg, pl.ds(j * L, L)], v)

    for s in range(SEG):
      for j in range(OUT // L):
        pacc[s, pl.ds(j * L, L)] = jnp.maximum(pacc[s, pl.ds(j * L, L)], 0.0)
    pltpu.sync_copy(pacc, out_hbm.at[pl.ds(_al(wid * SEG), SEG)])

  return k(rows2, batch)


def kernel(x, edge_index, batch, W1, a_src1, a_dst1, b1, W2, a_src2, a_dst2,
           b2, Wfc, bfc):
  src = edge_index[0]
  dst = edge_index[1]
  loop = jnp.arange(N, dtype=src.dtype)
  s_full = jnp.concatenate([src, loop])
  d_full = jnp.concatenate([dst, loop])

  x_pad = jnp.pad(x, ((0, NPAD - N), (0, 0)))
  w1r = W1.reshape(F_IN, H1, C1).transpose(1, 0, 2)
  as1 = a_src1.reshape(H1, 1, C1)
  ad1 = a_dst1.reshape(H1, 1, C1)
  b1r = b1.reshape(H1, 1, C1)
  w2r = W2.reshape(H1, C1, OUT)
  as2 = a_src2.reshape(1, 1, OUT)
  ad2 = a_dst2.reshape(1, 1, OUT)
  b2r = b2.reshape(1, OUT)
  bfcr = bfc.reshape(1, OUT)

  counts = _sc_bin_count(d_full)
  sbin, dbin = _sc_bin_fill(d_full, s_full, counts)

  h1T, als1, ald1 = _tc1(x_pad, w1r, as1, ad1)
  msg1, den1 = _sc_layer(H1, C1, als1.reshape(-1), ald1.reshape(-1), h1T,
                         counts, sbin, dbin)

  h2, als2, ald2 = _tc2(msg1, den1.reshape(H1, 1, NPAD), b1r, w2r, as2, ad2)
  h2t = h2.reshape(1, NPAD, OUT)
  msg2, den2 = _sc_layer(1, OUT, als2.reshape(-1), ald2.reshape(-1), h2t,
                         counts, sbin, dbin)

  rows2 = _tc3(msg2.reshape(NPAD, OUT), den2.reshape(1, 1, NPAD), b2r)
  pooled = _sc_pool(rows2.reshape(-1), batch)
  return _tc4(pooled, Wfc, bfcr)


# speedup vs baseline: 5.5403x; 1.5079x over previous
import dataclasses
import functools

import numpy as np

import jax
import jax.numpy as jnp
from jax import lax
from jax.experimental import pallas as pl
from jax.experimental.pallas import tpu as pltpu
from jax.experimental.pallas import tpu_sc as plsc

N = 10000
E = 160000
EN = E + N
F_IN = 256
H1 = 10
C1 = 256
OUT = 128
B = 512

NPAD = 10240
NT = 1024
NW = 32
R = 320
L = 16

CH0 = 34000
FLUSH = 1024
BE = 2016
G = 96
BINCAP = 173056

_MESH = dict(core_axis_name="c", subcore_axis_name="s")

_SC_PARAMS = pltpu.CompilerParams()
if "needs_layout_passes" in pltpu.CompilerParams.__dataclass_fields__:
  _SC_PARAMS = dataclasses.replace(_SC_PARAMS, needs_layout_passes=False)

_P32 = np.concatenate([np.arange(0, 32, 2), np.arange(1, 32, 2)])


def _perm(n):
  return np.concatenate([b * 32 + _P32 for b in range(n // 32)])


def _wid():
  return lax.axis_index("c") * 16 + lax.axis_index("s")


def _al(v):
  return pl.multiple_of(v, L)


def _tc1(x_pad, w1r, as1, ad1):
  def body(x_ref, w_ref, as_ref, ad_ref, h_ref, als_ref, ald_ref):
    hh = jnp.dot(x_ref[...], w_ref[0], preferred_element_type=jnp.float32)
    h_ref[0] = hh.astype(jnp.bfloat16)
    als_ref[0, 0] = jnp.sum(hh * as_ref[0], axis=1)
    ald_ref[0, 0] = jnp.sum(hh * ad_ref[0], axis=1)

  return pl.pallas_call(
      body,
      grid=(H1, NPAD // NT),
      in_specs=[
          pl.BlockSpec((NT, F_IN), lambda h, n: (n, 0)),
          pl.BlockSpec((1, F_IN, C1), lambda h, n: (h, 0, 0)),
          pl.BlockSpec((1, 1, C1), lambda h, n: (h, 0, 0)),
          pl.BlockSpec((1, 1, C1), lambda h, n: (h, 0, 0)),
      ],
      out_specs=[
          pl.BlockSpec((1, NT, C1), lambda h, n: (h, n, 0)),
          pl.BlockSpec((1, 1, NT), lambda h, n: (h, 0, n)),
          pl.BlockSpec((1, 1, NT), lambda h, n: (h, 0, n)),
      ],
      out_shape=[
          jax.ShapeDtypeStruct((H1, NPAD, C1), jnp.bfloat16),
          jax.ShapeDtypeStruct((H1, 1, NPAD), jnp.float32),
          jax.ShapeDtypeStruct((H1, 1, NPAD), jnp.float32),
      ],
  )(x_pad, w1r, as1, ad1)


def _tc2(msg1, den1, b1r, w2r, as2, ad2):
  def body(m_ref, d_ref, b_ref, w_ref, s_ref, t_ref, h2_ref, als_ref, ald_ref):
    acc = jnp.zeros((NT, OUT), jnp.float32)
    for h in range(H1):
      dh = d_ref[h, 0, :][:, None] + 1e-16
      hv = m_ref[h] / dh + b_ref[h]
      hp = jnp.where(hv > 0, hv, jnp.exp(jnp.minimum(hv, 0.0)) - 1.0)
      acc = acc + jnp.dot(hp, w_ref[h], preferred_element_type=jnp.float32)
    h2_ref[...] = acc
    als_ref[0, 0] = jnp.sum(acc * s_ref[0], axis=1)
    ald_ref[0, 0] = jnp.sum(acc * t_ref[0], axis=1)

  return pl.pallas_call(
      body,
      grid=(NPAD // NT,),
      in_specs=[
          pl.BlockSpec((H1, NT, C1), lambda n: (0, n, 0)),
          pl.BlockSpec((H1, 1, NT), lambda n: (0, 0, n)),
          pl.BlockSpec((H1, 1, C1), lambda n: (0, 0, 0)),
          pl.BlockSpec((H1, C1, OUT), lambda n: (0, 0, 0)),
          pl.BlockSpec((1, 1, OUT), lambda n: (0, 0, 0)),
          pl.BlockSpec((1, 1, OUT), lambda n: (0, 0, 0)),
      ],
      out_specs=[
          pl.BlockSpec((NT, OUT), lambda n: (n, 0)),
          pl.BlockSpec((1, 1, NT), lambda n: (0, 0, n)),
          pl.BlockSpec((1, 1, NT), lambda n: (0, 0, n)),
      ],
      out_shape=[
          jax.ShapeDtypeStruct((NPAD, OUT), jnp.float32),
          jax.ShapeDtypeStruct((1, 1, NPAD), jnp.float32),
          jax.ShapeDtypeStruct((1, 1, NPAD), jnp.float32),
      ],
  )(msg1, den1, b1r, w2r, as2, ad2)


def _tc3(msg2, den2, b2r):
  def body(m_ref, d_ref, b_ref, o_ref):
    o_ref[...] = jnp.maximum(
        m_ref[...] / (d_ref[0, 0, :][:, None] + 1e-16) + b_ref[...], 0.0)

  return pl.pallas_call(
      body,
      grid=(NPAD // NT,),
      in_specs=[
          pl.BlockSpec((NT, OUT), lambda n: (n, 0)),
          pl.BlockSpec((1, 1, NT), lambda n: (0, 0, n)),
          pl.BlockSpec((1, OUT), lambda n: (0, 0)),
      ],
      out_specs=pl.BlockSpec((NT, OUT), lambda n: (n, 0)),
      out_shape=jax.ShapeDtypeStruct((NPAD, OUT), jnp.float32),
  )(msg2, den2, b2r)


def _tc4(pooled, wfc, bfcr):
  def body(p_ref, w_ref, b_ref, o_ref):
    o_ref[...] = jnp.maximum(
        jnp.dot(p_ref[...], w_ref[...], preferred_element_type=jnp.float32)
        + b_ref[...], 0.0)

  return pl.pallas_call(
      body,
      out_shape=jax.ShapeDtypeStruct((B, OUT), jnp.float32),
  )(pooled, wfc, bfcr)


def _sc_bin_count(d_full):
  @functools.partial(
      pl.kernel,
      out_type=jax.ShapeDtypeStruct((NW, L), jnp.int32),
      mesh=plsc.VectorSubcoreMesh(**_MESH),
      compiler_params=_SC_PARAMS,
      scratch_types=[
          pltpu.VMEM((CH0,), jnp.int32),
          pltpu.VMEM((L,), jnp.int32),
          pltpu.SemaphoreType.DMA,
      ],
  )
  def k(d_hbm, counts_hbm, dbuf, cbuf, sem):
    wid = _wid()
    lo = wid * R
    hi = lo + R

    def chunk(ci, cnt):
      pltpu.sync_copy(d_hbm.at[pl.ds(ci * CH0, CH0)], dbuf)

      def vec(vi, cnt):
        d16 = dbuf[pl.ds(vi * L, L)]
        m = (d16 >= lo) & (d16 < hi)
        return cnt + jnp.where(m, 1, 0).astype(jnp.int32)

      return lax.fori_loop(0, CH0 // L, vec, cnt)

    cnt = lax.fori_loop(0, EN // CH0, chunk, jnp.zeros((L,), jnp.int32))
    total = jnp.sum(cnt)
    cbuf[...] = jnp.full((L,), total, jnp.int32)
    pltpu.sync_copy(cbuf, counts_hbm.at[wid])

  return k(d_full)


def _prefix_from_counts(cbuf, wid):
  my_start = jnp.int32(0)
  total = jnp.int32(0)
  my_count = jnp.int32(0)
  for rr in range(NW):
    c = jnp.max(cbuf[rr])
    cpad = ((c + L - 1) // L) * L
    my_start = my_start + jnp.where(rr < wid, cpad, 0)
    my_count = my_count + jnp.where(rr == wid, c, 0)
    total = total + cpad
  return my_start, my_count, total


def _sc_bin_fill(d_full, s_full, counts):
  @functools.partial(
      pl.kernel,
      out_type=[
          jax.ShapeDtypeStruct((BINCAP,), jnp.int32),
          jax.ShapeDtypeStruct((BINCAP,), jnp.int32),
      ],
      mesh=plsc.VectorSubcoreMesh(**_MESH),
      compiler_params=_SC_PARAMS,
      scratch_types=[
          pltpu.VMEM((CH0,), jnp.int32),
          pltpu.VMEM((CH0,), jnp.int32),
          pltpu.VMEM((FLUSH + L,), jnp.int32),
          pltpu.VMEM((FLUSH + L,), jnp.int32),
          pltpu.VMEM((NW, L), jnp.int32),
          pltpu.SemaphoreType.DMA,
      ],
  )
  def k(d_hbm, s_hbm, c_hbm, sb_hbm, db_hbm, dbuf, sbuf, sts, std, cbuf, sem):
    wid = _wid()
    lo = wid * R
    hi = lo + R
    pltpu.sync_copy(c_hbm, cbuf)
    my_start, my_count, total = _prefix_from_counts(cbuf, wid)

    zero = jnp.zeros((L,), jnp.int32)
    for z in range((FLUSH + L) // L):
      sts[pl.ds(z * L, L)] = zero
      std[pl.ds(z * L, L)] = zero

    def chunk(ci, carry):
      pltpu.sync_copy(d_hbm.at[pl.ds(ci * CH0, CH0)], dbuf)
      pltpu.sync_copy(s_hbm.at[pl.ds(ci * CH0, CH0)], sbuf)

      def vec(vi, carry):
        staged, cursor = carry
        d16 = dbuf[pl.ds(vi * L, L)]
        s16 = sbuf[pl.ds(vi * L, L)]
        m = (d16 >= lo) & (d16 < hi)
        c = jnp.sum(jnp.where(m, 1, 0).astype(jnp.int32))
        plsc.store_compressed(sts.at[pl.ds(staged, L)], s16, mask=m)
        plsc.store_compressed(std.at[pl.ds(staged, L)], d16 - lo, mask=m)
        staged = staged + c

        def flush(carry):
          staged, cursor = carry
          pltpu.sync_copy(sts.at[pl.ds(0, FLUSH)],
                          sb_hbm.at[pl.ds(_al(cursor), FLUSH)])
          pltpu.sync_copy(std.at[pl.ds(0, FLUSH)],
                          db_hbm.at[pl.ds(_al(cursor), FLUSH)])
          rs = sts[pl.ds(FLUSH, L)]
          rd = std[pl.ds(FLUSH, L)]
          sts[pl.ds(0, L)] = rs
          std[pl.ds(0, L)] = rd
          return staged - FLUSH, cursor + FLUSH

        return lax.cond(staged >= FLUSH, flush, lambda c: c, (staged, cursor))

      return lax.fori_loop(0, CH0 // L, vec, carry)

    staged, cursor = lax.fori_loop(0, EN // CH0, chunk, (jnp.int32(0), my_start))

    ngr = (staged + L - 1) // L

    def drain(gi, _):
      pltpu.sync_copy(sts.at[pl.ds(gi * L, L)],
                      sb_hbm.at[pl.ds(_al(cursor + gi * L), L)])
      pltpu.sync_copy(std.at[pl.ds(gi * L, L)],
                      db_hbm.at[pl.ds(_al(cursor + gi * L), L)])
      return 0

    lax.fori_loop(0, ngr, drain, 0)

    @pl.when(wid == NW - 1)
    def _():
      for z in range(FLUSH // L):
        sts[pl.ds(z * L, L)] = zero
      pltpu.sync_copy(sts.at[pl.ds(0, FLUSH)],
                      sb_hbm.at[pl.ds(_al(total), FLUSH)])
      pltpu.sync_copy(sts.at[pl.ds(0, FLUSH)],
                      sb_hbm.at[pl.ds(_al(total + FLUSH), FLUSH)])
      pltpu.sync_copy(sts.at[pl.ds(0, FLUSH)],
                      db_hbm.at[pl.ds(_al(total), FLUSH)])
      pltpu.sync_copy(sts.at[pl.ds(0, FLUSH)],
                      db_hbm.at[pl.ds(_al(total + FLUSH), FLUSH)])

  return k(d_full, s_full, counts)


def _sc_layer(heads, width, packed, alpha_sT, alpha_dT, rows_tab, counts,
              sbin, dbin):
  jc = width // L

  @functools.partial(
      pl.kernel,
      out_type=[
          jax.ShapeDtypeStruct((heads, NPAD, width), jnp.float32),
          jax.ShapeDtypeStruct((heads * NPAD,), jnp.float32),
      ],
      mesh=plsc.VectorSubcoreMesh(**_MESH),
      compiler_params=_SC_PARAMS,
      scratch_types=[
          pltpu.VMEM((NPAD,), jnp.float32),
          pltpu.VMEM((R,), jnp.float32),
          pltpu.VMEM((R, width), jnp.float32),
          pltpu.VMEM((G, width // 2) if packed else (G, width),
                     jnp.int32 if packed else jnp.float32),
          pltpu.VMEM((G, width // 2) if packed else (G, width),
                     jnp.int32 if packed else jnp.float32),
          pltpu.VMEM((BE,), jnp.int32),
          pltpu.VMEM((BE,), jnp.int32),
          pltpu.VMEM((R,), jnp.float32),
          pltpu.VMEM((NW, L), jnp.int32),
          pltpu.SemaphoreType.DMA,
          pltpu.SemaphoreType.DMA,
          pltpu.SemaphoreType.DMA,
      ],
  )
  def k(as_hbm, ad_hbm, tab_hbm, c_hbm, sb_hbm, db_hbm, msg_hbm, den_hbm,
        table, adr, acc, rowsA, rowsB, sblk, dblk, dn, cbuf, semA, semB, sem):
    wid = _wid()
    lo = wid * R
    pltpu.sync_copy(c_hbm, cbuf)
    my_start, my_count, _ = _prefix_from_counts(cbuf, wid)
    nblk = (my_count + BE - 1) // BE
    iota = lax.iota(jnp.int32, L)

    def gather(h, c, dst, dsem):
      return pltpu.make_async_copy(
          tab_hbm.at[h].at[sblk.at[pl.ds(c * G, G)]], dst, dsem)

    def per_head(h):
      pltpu.sync_copy(as_hbm.at[pl.ds(_al(h * NPAD), NPAD)], table)
      pltpu.sync_copy(ad_hbm.at[pl.ds(_al(h * NPAD + lo), R)], adr)

      def tmax(i, m):
        return jnp.maximum(m, table[pl.ds(i * L, L)])

      gmax = jnp.max(lax.fori_loop(0, N // L, tmax,
                                   jnp.full((L,), -3e38, jnp.float32)))

      zf = jnp.zeros((L,), jnp.float32)

      def zrow(r, _):
        for j in range(jc):
          acc[r, pl.ds(j * L, L)] = zf
        return 0

      lax.fori_loop(0, R, zrow, 0)
      for z in range(R // L):
        dn[pl.ds(z * L, L)] = zf

      def per_block(bi):
        eb0 = bi * BE
        ebn = jnp.minimum(BE, my_count - eb0)
        pltpu.sync_copy(sb_hbm.at[pl.ds(_al(my_start + eb0), BE)], sblk)
        pltpu.sync_copy(db_hbm.at[pl.ds(_al(my_start + eb0), BE)], dblk)
        nch = (ebn + G - 1) // G
        gather(h, 0, rowsA, semA).start()

        def process(c, rows):
          @pl.loop(0, G // L)
          def _(g2):
            base = c * G + g2 * L
            s16 = sblk[pl.ds(base, L)]
            dl16 = dblk[pl.ds(base, L)]
            as16 = plsc.load_gather(table, [s16])
            ad16 = plsc.load_gather(adr, [dl16])
            e16 = as16 + ad16
            e16 = jnp.where(e16 > 0, e16, 0.2 * e16)
            t16 = gmax + ad16
            m16 = jnp.where(t16 > 0, t16, 0.2 * t16)
            w16 = jnp.exp(e16 - m16)
            w16 = jnp.where(base + iota < ebn, w16, 0.0)
            plsc.addupdate_scatter(dn, [dl16], w16)
            for li in range(L):
              w_s = w16[li]
              dl_s = dl16[li]
              if packed:
                for j in range(width // 32):
                  wv = rows[g2 * L + li, pl.ds(j * L, L)]
                  ve = plsc.bitcast(wv << 16, jnp.float32)
                  vo = plsc.bitcast(wv & jnp.int32(-65536), jnp.float32)
                  acc[dl_s, pl.ds(j * 32, L)] += w_s * ve
                  acc[dl_s, pl.ds(j * 32 + L, L)] += w_s * vo
              else:
                for j in range(width // L):
                  acc[dl_s, pl.ds(j * L, L)] += w_s * rows[g2 * L + li,
                                                           pl.ds(j * L, L)]

        def pair(kk, _):
          c0 = 2 * kk
          c1 = c0 + 1

          @pl.when(c1 < nch)
          def _():
            gather(h, c1, rowsB, semB).start()

          gather(h, c0, rowsA, semA).wait()
          process(c0, rowsA)

          @pl.when(c1 < nch)
          def _():
            @pl.when(c1 + 1 < nch)
            def _():
              gather(h, c1 + 1, rowsA, semA).start()

            gather(h, c1, rowsB, semB).wait()
            process(c1, rowsB)

          return 0

        lax.fori_loop(0, (nch + 1) // 2, pair, 0)

      @pl.loop(0, nblk)
      def _(bi):
        per_block(bi)

      pltpu.sync_copy(acc, msg_hbm.at[h].at[pl.ds(_al(lo), R)])
      pltpu.sync_copy(dn, den_hbm.at[pl.ds(_al(h * NPAD + lo), R)])

    @pl.loop(0, heads)
    def _(h):
      per_head(h)

  return k(alpha_sT, alpha_dT, rows_tab, counts, sbin, dbin)


def _sc_pool(rows2, batch):
  PC = 256
  SEG = B // NW

  @functools.partial(
      pl.kernel,
      out_type=jax.ShapeDtypeStruct((B, OUT), jnp.float32),
      mesh=plsc.VectorSubcoreMesh(**_MESH),
      compiler_params=_SC_PARAMS,
      scratch_types=[
          pltpu.VMEM((N + PC,), jnp.int32),
          pltpu.VMEM((PC * OUT,), jnp.float32),
          pltpu.VMEM((SEG, OUT), jnp.float32),
          pltpu.SemaphoreType.DMA,
      ],
  )
  def k(rows_hbm, b_hbm, out_hbm, bbuf, rbuf, pacc, sem):
    wid = _wid()
    seg0 = wid * SEG
    zi = jnp.zeros((L,), jnp.int32)
    for z in range(PC // L):
      bbuf[pl.ds(N + z * L, L)] = zi
    pltpu.sync_copy(b_hbm, bbuf.at[pl.ds(0, N)])

    def cnt(i, carry):
      cl, chh = carry
      b16 = bbuf[pl.ds(i * L, L)]
      cl = cl + jnp.where(b16 < seg0, 1, 0).astype(jnp.int32)
      chh = chh + jnp.where(b16 < seg0 + SEG, 1, 0).astype(jnp.int32)
      return cl, chh

    cl, chh = lax.fori_loop(0, N // L, cnt, (zi, zi))
    lo = jnp.sum(cl)
    hi = jnp.sum(chh)

    neg = jnp.full((L,), -1.0, jnp.float32)
    for s in range(SEG):
      for j in range(OUT // L):
        pacc[s, pl.ds(j * L, L)] = neg

    nck = (hi - lo + PC - 1) // PC

    @pl.loop(0, nck)
    def _(ck):
      row0 = jnp.minimum(lo + ck * PC, NPAD - PC)
      pltpu.sync_copy(rows_hbm.at[pl.ds(_al(row0 * OUT), PC * OUT)], rbuf)

      @pl.loop(0, PC // L)
      def _(g):
        b16 = jnp.clip(bbuf[pl.ds(row0 + g * L, L)] - seg0, 0, SEG - 1)
        for li in range(L):
          rr = g * L + li
          valid = (row0 + rr >= lo) & (row0 + rr < hi)
          sg = b16[li]
          for j in range(OUT // L):
            v = jnp.where(valid, rbuf[pl.ds(rr * OUT + j * L, L)], -1.0)
            pacc[sg, pl.ds(j * L, L)] = jnp.maximum(pacc[sg, pl.ds(j * L, L)], v)

    for s in range(SEG):
      for j in range(OUT // L):
        pacc[s, pl.ds(j * L, L)] = jnp.maximum(pacc[s, pl.ds(j * L, L)], 0.0)
    pltpu.sync_copy(pacc, out_hbm.at[pl.ds(_al(wid * SEG), SEG)])

  return k(rows2, batch)


def kernel(x, edge_index, batch, W1, a_src1, a_dst1, b1, W2, a_src2, a_dst2,
           b2, Wfc, bfc):
  src = edge_index[0]
  dst = edge_index[1]
  loop = jnp.arange(N, dtype=src.dtype)
  s_full = jnp.concatenate([src, loop])
  d_full = jnp.concatenate([dst, loop])

  x_pad = jnp.pad(x, ((0, NPAD - N), (0, 0)))
  w1r = W1.reshape(F_IN, H1, C1).transpose(1, 0, 2)
  as1 = a_src1.reshape(H1, 1, C1)
  ad1 = a_dst1.reshape(H1, 1, C1)
  pc = _perm(C1)
  po = _perm(OUT)
  b1r = b1.reshape(H1, 1, C1)[:, :, pc]
  w2r = W2.reshape(H1, C1, OUT)[:, pc, :]
  as2 = a_src2.reshape(1, 1, OUT)
  ad2 = a_dst2.reshape(1, 1, OUT)
  b2r = b2.reshape(1, OUT)
  bfcr = bfc.reshape(1, OUT)

  counts = _sc_bin_count(d_full)
  sbin, dbin = _sc_bin_fill(d_full, s_full, counts)

  h1T, als1, ald1 = _tc1(x_pad, w1r, as1, ad1)
  h1i = lax.bitcast_convert_type(
      h1T.reshape(H1, NPAD, C1 // 2, 2), jnp.int32)
  msg1, den1 = _sc_layer(H1, C1, True, als1.reshape(-1), ald1.reshape(-1),
                         h1i, counts, sbin, dbin)

  h2, als2, ald2 = _tc2(msg1, den1.reshape(H1, 1, NPAD), b1r, w2r, as2, ad2)
  h2t = h2.reshape(1, NPAD, OUT)
  msg2, den2 = _sc_layer(1, OUT, False, als2.reshape(-1), ald2.reshape(-1),
                         h2t, counts, sbin, dbin)

  rows2 = _tc3(msg2.reshape(NPAD, OUT), den2.reshape(1, 1, NPAD), b2r)
  pooled = _sc_pool(rows2.reshape(-1), batch)
  return _tc4(pooled, Wfc, bfcr)
